# baseline (device time: 387319 ns/iter reference)
import jax
import jax.numpy as jnp
from jax import lax
from jax.experimental import pallas as pl
from jax.experimental.pallas import tpu as pltpu

M = 4096
N = 8192
K_LOC = 4096
HALF = M // 2
QTR = HALF // 4

MC = 3072

N_BLK = 1024
K_BLK = 512
J_STEPS = N // N_BLK
K_STEPS = K_LOC // K_BLK


def _ring(my_x, my_y, my_z):
    r = 2 * my_x + (my_z ^ my_x)
    opp = (r + 2) % 4
    left_r = (r + 3) % 4
    right_r = (r + 1) % 4
    r_even = (r % 2) == 0
    right_x = jnp.where(r_even, my_x, 1 - my_x)
    right_z = jnp.where(r_even, 1 - my_z, my_z)
    left_x = jnp.where(r_even, 1 - my_x, my_x)
    left_z = jnp.where(r_even, my_z, 1 - my_z)
    right = (right_x, my_y, right_z)
    left = (left_x, my_y, left_z)
    return r, opp, left_r, right_r, left, right


def _prep_body(cols_ref, x_ref, o_ref):
    del cols_ref
    o_ref[...] = x_ref[...].astype(jnp.bfloat16)


def _prep(cols, x):
    grid = (MC // 512, K_LOC // 512)
    return pl.pallas_call(
        _prep_body,
        grid_spec=pltpu.PrefetchScalarGridSpec(
            num_scalar_prefetch=1,
            grid=grid,
            in_specs=[
                pl.BlockSpec((512, 512), lambda c, kb, cols: (kb, cols[c])),
            ],
            out_specs=pl.BlockSpec((512, 512), lambda c, kb, cols: (kb, c)),
        ),
        out_shape=jax.ShapeDtypeStruct((K_LOC, MC), jnp.bfloat16),
    )(cols, x)


def _main_body(x_ref, dy_ref, mine_ref, recv_ref, acc_ref, sendbuf,
               as_s, as_r, fw_s, fw_r):
    j = pl.program_id(0)
    k = pl.program_id(1)
    my_x = lax.axis_index("x")
    my_y = lax.axis_index("y")
    my_z = lax.axis_index("z")
    other = 1 - my_y
    r, opp, left_r, right_r, left, right = _ring(my_x, my_y, my_z)
    y_nbr = (my_x, other, my_z)

    def band(b):
        return pl.ds(b * N_BLK, N_BLK)

    def slot(s, b):
        return recv_ref.at[pl.ds(s * QTR, QTR), band(b)]

    def a_rdma(q, b):
        dst_slot = r if q == 0 else opp
        return pltpu.make_async_remote_copy(
            src_ref=sendbuf.at[q, b % 2],
            dst_ref=slot(dst_slot, b),
            send_sem=as_s.at[q, b],
            recv_sem=as_r.at[q, b],
            device_id=y_nbr,
            device_id_type=pl.DeviceIdType.MESH,
        )

    def fw_rdma(d, b):
        return pltpu.make_async_remote_copy(
            src_ref=slot(r, b),
            dst_ref=slot(r, b),
            send_sem=fw_s.at[d, b],
            recv_sem=fw_r.at[d, b],
            device_id=right if d == 0 else left,
            device_id_type=pl.DeviceIdType.MESH,
        )

    def fw_wait(d, b):
        return pltpu.make_async_remote_copy(
            src_ref=slot(r, b),
            dst_ref=slot(left_r if d == 0 else right_r, b),
            send_sem=fw_s.at[d, b],
            recv_sem=fw_r.at[d, b],
            device_id=left if d == 0 else right,
            device_id_type=pl.DeviceIdType.MESH,
        )

    @pl.when(k == 0)
    def _():
        acc_ref[...] = jnp.zeros_like(acc_ref)

    acc_ref[...] += lax.dot_general(
        x_ref[pl.ds(k * K_BLK, K_BLK), :], dy_ref[...].astype(jnp.bfloat16),
        (((0,), (0,)), ((), ())),
        preferred_element_type=jnp.float32,
    )

    @pl.when(k == K_STEPS - 1)
    def _():
        mine_ref[...] = acc_ref[2 * QTR:, :].astype(jnp.bfloat16)

        @pl.when(j >= 2)
        def _():
            a_rdma(0, j - 2).wait_send()
            a_rdma(1, j - 2).wait_send()

        sendbuf[0, j % 2] = acc_ref[0:QTR, :].astype(jnp.bfloat16)
        sendbuf[1, j % 2] = acc_ref[QTR:2 * QTR, :].astype(jnp.bfloat16)
        a0 = a_rdma(0, j)
        a0.start()
        a1 = a_rdma(1, j)
        a1.start()

    @pl.when(jnp.logical_and(j > 0, k == K_STEPS // 2))
    def _():
        b = j - 1
        a_rdma(0, b).wait_recv()
        fw_rdma(0, b).start()
        fw_rdma(1, b).start()

    @pl.when(jnp.logical_and(j == J_STEPS - 1, k == K_STEPS - 1))
    def _():
        b = J_STEPS - 1
        a_rdma(0, b).wait_recv()
        fw_rdma(0, b).start()
        fw_rdma(1, b).start()
        for bb in range(J_STEPS):
            a_rdma(1, bb).wait_recv()
            fw_wait(0, bb).wait_recv()
            fw_wait(1, bb).wait_recv()
            fw_rdma(0, bb).wait_send()
            fw_rdma(1, bb).wait_send()
        for bb in (J_STEPS - 2, J_STEPS - 1):
            a_rdma(0, bb).wait_send()
            a_rdma(1, bb).wait_send()


def _main(x_cat, dy):
    grid = (J_STEPS, K_STEPS)
    return pl.pallas_call(
        _main_body,
        grid=grid,
        in_specs=[
            pl.BlockSpec((K_LOC, MC), lambda j, k: (0, 0)),
            pl.BlockSpec((K_BLK, N_BLK), lambda j, k: (k, j)),
        ],
        out_specs=[
            pl.BlockSpec((HALF, N_BLK), lambda j, k: (0, j)),
            pl.BlockSpec(memory_space=pl.ANY),
        ],
        out_shape=[
            jax.ShapeDtypeStruct((HALF, N), jnp.bfloat16),
            jax.ShapeDtypeStruct((HALF, N), jnp.bfloat16),
        ],
        scratch_shapes=[
            pltpu.VMEM((MC, N_BLK), jnp.float32),
            pltpu.VMEM((2, 2, QTR, N_BLK), jnp.bfloat16),
            pltpu.SemaphoreType.DMA((2, J_STEPS)),
            pltpu.SemaphoreType.DMA((2, J_STEPS)),
            pltpu.SemaphoreType.DMA((2, J_STEPS)),
            pltpu.SemaphoreType.DMA((2, J_STEPS)),
        ],
        compiler_params=pltpu.CompilerParams(
            dimension_semantics=("arbitrary", "arbitrary"),
            vmem_limit_bytes=100 * 1024 * 1024,
        ),
    )(x_cat, dy)


def _add_body(a_ref, b_ref, o_ref):
    o_ref[...] = a_ref[...].astype(jnp.float32) + b_ref[...].astype(jnp.float32)


_ADD_BLK = 128


def _add(a, b):
    grid = (HALF // _ADD_BLK,)
    spec = pl.BlockSpec((_ADD_BLK, N), lambda i: (i, 0))
    return pl.pallas_call(
        _add_body,
        grid=grid,
        in_specs=[spec, spec],
        out_specs=pl.BlockSpec((_ADD_BLK, N), lambda i: (i, 0)),
        out_shape=jax.ShapeDtypeStruct((HALF, N), jnp.float32),
    )(a, b)


def kernel(x, dy):
    my_x = lax.axis_index("x")
    my_y = lax.axis_index("y")
    my_z = lax.axis_index("z")
    other = 1 - my_y
    r, opp, _, _, _, _ = _ring(my_x, my_y, my_z)
    cols = jnp.stack([
        other * 4 + r,
        other * 4 + opp,
        my_y * 4 + 0,
        my_y * 4 + 1,
        my_y * 4 + 2,
        my_y * 4 + 3,
    ]).astype(jnp.int32)
    x_cat = _prep(cols, x)
    mine, recv = _main(x_cat, dy)
    return _add(mine, recv)
